# baseline (speedup 1.0000x reference)
import functools
import jax
import jax.numpy as jnp
from jax import lax
from jax.experimental import pallas as pl
from jax.experimental.pallas import tpu as pltpu
from jax.experimental.pallas import tpu_sc as plsc


def _make_sc_copy(S, H, dtype):
    info = plsc.get_sparse_core_info()
    NC, NS = info.num_cores, info.num_subcores
    NW = NC * NS
    rows_per_w = S // NW
    mesh = plsc.VectorSubcoreMesh(core_axis_name="c", subcore_axis_name="s")

    @functools.partial(
        pl.kernel,
        out_type=jax.ShapeDtypeStruct((1, S, H), dtype),
        mesh=mesh,
        scratch_types=[pltpu.VMEM((rows_per_w, H), dtype)],
    )
    def sc_copy(src_hbm, out_hbm, buf_v):
        wid = lax.axis_index("s") * NC + lax.axis_index("c")
        base = wid * rows_per_w
        pltpu.sync_copy(src_hbm.at[0, pl.ds(base, rows_per_w), :], buf_v)
        pltpu.sync_copy(buf_v, out_hbm.at[0, pl.ds(base, rows_per_w), :])

    return sc_copy


def kernel(last_hidden_state, input_ids):
    del input_ids
    B, S, H = last_hidden_state.shape
    return _make_sc_copy(S, H, last_hidden_state.dtype)(last_hidden_state)
